# baseline (speedup 1.0000x reference)
import functools

import jax
import jax.numpy as jnp
from jax.experimental import pallas as pl
from jax.experimental.pallas import tpu as pltpu

_KS = 2
_BN_EPS = 1e-5


def _conv_stats_kernel(p_ref, w_ref, b_ref, y_ref, sum_ref, ssq_ref):
    y = jnp.dot(w_ref[...], p_ref[...], preferred_element_type=jnp.float32)
    y = y + b_ref[...]
    y_ref[...] = y
    sum_ref[...] = jnp.sum(y, axis=1, keepdims=True)
    ssq_ref[...] = jnp.sum(y * y, axis=1, keepdims=True)


def _bn_prelu_kernel(y_ref, scale_ref, shift_ref, alpha_ref, o_ref):
    z = y_ref[...] * scale_ref[...] + shift_ref[...]
    o_ref[...] = jnp.where(z > 0, z, alpha_ref[...] * z)


def kernel(x, conv_w, conv_b, bn_gamma, bn_beta, prelu_alpha):
    N, Cin, D, H, W = x.shape
    Cout = conv_w.shape[0]
    Do, Ho, Wo = D // _KS, H // _KS, W // _KS
    spatial = Do * Ho * Wo
    kdim = _KS * _KS * _KS * Cin

    xt = x.reshape(N, Cin, Do, _KS, Ho, _KS, Wo, _KS)
    patches = jnp.transpose(xt, (0, 1, 3, 5, 7, 2, 4, 6))
    patches = patches.reshape(N, kdim, spatial).astype(jnp.bfloat16)

    w_mat = conv_w.reshape(Cout, kdim).astype(jnp.bfloat16)
    b_col = conv_b.reshape(Cout, 1)

    tile_s = min(spatial, 4096)
    grid_s = spatial // tile_s

    y_t, psum, pssq = pl.pallas_call(
        _conv_stats_kernel,
        out_shape=(
            jax.ShapeDtypeStruct((N, Cout, spatial), jnp.float32),
            jax.ShapeDtypeStruct((N * grid_s, Cout, 1), jnp.float32),
            jax.ShapeDtypeStruct((N * grid_s, Cout, 1), jnp.float32),
        ),
        grid=(N, grid_s),
        in_specs=[
            pl.BlockSpec((None, kdim, tile_s), lambda n, s: (n, 0, s)),
            pl.BlockSpec((Cout, kdim), lambda n, s: (0, 0)),
            pl.BlockSpec((Cout, 1), lambda n, s: (0, 0)),
        ],
        out_specs=(
            pl.BlockSpec((None, Cout, tile_s), lambda n, s: (n, 0, s)),
            pl.BlockSpec((None, Cout, 1), lambda n, s, gs=grid_s: (n * gs + s, 0, 0)),
            pl.BlockSpec((None, Cout, 1), lambda n, s, gs=grid_s: (n * gs + s, 0, 0)),
        ),
        compiler_params=pltpu.CompilerParams(
            dimension_semantics=("parallel", "parallel")),
    )(patches, w_mat, b_col)

    cnt = jnp.float32(N * spatial)
    s = jnp.sum(psum, axis=(0, 2))
    sq = jnp.sum(pssq, axis=(0, 2))
    mean = s / cnt
    var = jnp.maximum(sq / cnt - mean * mean, 0.0)
    inv = jax.lax.rsqrt(var + _BN_EPS)
    scale = (bn_gamma * inv).reshape(Cout, 1)
    shift = (bn_beta - mean * bn_gamma * inv).reshape(Cout, 1)

    out_t = pl.pallas_call(
        _bn_prelu_kernel,
        out_shape=jax.ShapeDtypeStruct((N, Cout, spatial), jnp.float32),
        grid=(N, grid_s),
        in_specs=[
            pl.BlockSpec((None, Cout, tile_s), lambda n, s: (n, 0, s)),
            pl.BlockSpec((Cout, 1), lambda n, s: (0, 0)),
            pl.BlockSpec((Cout, 1), lambda n, s: (0, 0)),
            pl.BlockSpec((1, 1), lambda n, s: (0, 0)),
        ],
        out_specs=pl.BlockSpec((None, Cout, tile_s), lambda n, s: (n, 0, s)),
        compiler_params=pltpu.CompilerParams(
            dimension_semantics=("parallel", "parallel")),
    )(y_t, scale, shift, prelu_alpha)

    return out_t.reshape(N, Cout, Do, Ho, Wo)
